# Initial kernel scaffold; baseline (speedup 1.0000x reference)
import jax
import jax.numpy as jnp
from jax.experimental import pallas as pl


def kernel(x, W1, W2):
    raise NotImplementedError("write your pallas kernel here")


# speedup vs baseline: 13.0297x; 13.0297x over previous
import functools

import jax
import jax.numpy as jnp
from jax.experimental import pallas as pl
from jax.experimental.pallas import tpu as pltpu


def _fused_gate_kernel(x_ref, w1t_ref, w2t_ref, o_ref):
    xb = x_ref[...]
    h = jax.lax.dot_general(
        xb, w1t_ref[...], (((1,), (0,)), ((), ())),
        preferred_element_type=jnp.float32,
    )
    h = jnp.maximum(h, 0.0)
    g = jax.lax.dot_general(
        h, w2t_ref[...], (((1,), (0,)), ((), ())),
        preferred_element_type=jnp.float32,
    )
    o_ref[...] = xb * jax.nn.sigmoid(g + g)


@functools.partial(jax.jit, static_argnames=("block_rows",))
def _run(x, w1t, w2t, block_rows):
    n, d = x.shape
    grid = (n // block_rows,)
    return pl.pallas_call(
        _fused_gate_kernel,
        grid=grid,
        in_specs=[
            pl.BlockSpec((block_rows, d), lambda i: (i, 0)),
            pl.BlockSpec((w1t.shape[0], w1t.shape[1]), lambda i: (0, 0)),
            pl.BlockSpec((w2t.shape[0], w2t.shape[1]), lambda i: (0, 0)),
        ],
        out_specs=pl.BlockSpec((block_rows, d), lambda i: (i, 0)),
        out_shape=jax.ShapeDtypeStruct((n, d), x.dtype),
        compiler_params=pltpu.CompilerParams(
            dimension_semantics=("arbitrary",),
        ),
    )(x, w1t, w2t)


def kernel(x, W1, W2):
    return _run(x, W1.T, W2.T, 1600)


# speedup vs baseline: 13.0356x; 1.0005x over previous
import functools

import jax
import jax.numpy as jnp
from jax.experimental import pallas as pl
from jax.experimental.pallas import tpu as pltpu


def _fused_gate_kernel(x_ref, w1t_ref, w2t_ref, o_ref):
    xb = x_ref[...]
    h = jax.lax.dot_general(
        xb, w1t_ref[...], (((1,), (0,)), ((), ())),
        preferred_element_type=jnp.float32,
    )
    h = jnp.maximum(h, 0.0)
    g = jax.lax.dot_general(
        h, w2t_ref[...], (((1,), (0,)), ((), ())),
        preferred_element_type=jnp.float32,
    )
    o_ref[...] = xb * jax.nn.sigmoid(g + g)


@functools.partial(jax.jit, static_argnames=("block_rows",))
def _run(x, w1t, w2t, block_rows):
    n, d = x.shape
    grid = (n // block_rows,)
    return pl.pallas_call(
        _fused_gate_kernel,
        grid=grid,
        in_specs=[
            pl.BlockSpec((block_rows, d), lambda i: (i, 0)),
            pl.BlockSpec((w1t.shape[0], w1t.shape[1]), lambda i: (0, 0)),
            pl.BlockSpec((w2t.shape[0], w2t.shape[1]), lambda i: (0, 0)),
        ],
        out_specs=pl.BlockSpec((block_rows, d), lambda i: (i, 0)),
        out_shape=jax.ShapeDtypeStruct((n, d), x.dtype),
        compiler_params=pltpu.CompilerParams(
            dimension_semantics=("parallel",),
        ),
    )(x, w1t, w2t)


def kernel(x, W1, W2):
    return _run(x, W1.T, W2.T, 1600)


# speedup vs baseline: 18.4397x; 1.4146x over previous
import functools

import jax
import jax.numpy as jnp
from jax.experimental import pallas as pl
from jax.experimental.pallas import tpu as pltpu


def _fused_gate_kernel(x_ref, w1t_ref, w2t_ref, o_ref):
    xb = x_ref[...]
    h = jax.lax.dot_general(
        xb, w1t_ref[...], (((1,), (0,)), ((), ())),
        preferred_element_type=jnp.float32,
    )
    h = jnp.maximum(h, 0.0)
    g = jax.lax.dot_general(
        h, w2t_ref[...], (((1,), (0,)), ((), ())),
        preferred_element_type=jnp.float32,
    )
    o_ref[...] = xb * jax.nn.sigmoid(g + g)


@functools.partial(jax.jit, static_argnames=("block_rows",))
def _run(x, w1t, w2t, block_rows):
    n, d = x.shape
    grid = (n // block_rows,)
    return pl.pallas_call(
        _fused_gate_kernel,
        grid=grid,
        in_specs=[
            pl.BlockSpec((block_rows, d), lambda i: (i, 0)),
            pl.BlockSpec((w1t.shape[0], w1t.shape[1]), lambda i: (0, 0)),
            pl.BlockSpec((w2t.shape[0], w2t.shape[1]), lambda i: (0, 0)),
        ],
        out_specs=pl.BlockSpec((block_rows, d), lambda i: (i, 0)),
        out_shape=jax.ShapeDtypeStruct((n, d), x.dtype),
        compiler_params=pltpu.CompilerParams(
            dimension_semantics=("parallel",),
        ),
    )(x, w1t, w2t)


def kernel(x, W1, W2):
    return _run(x, W1.T, W2.T, 3200)


# speedup vs baseline: 23.8010x; 1.2907x over previous
import functools

import jax
import jax.numpy as jnp
from jax.experimental import pallas as pl
from jax.experimental.pallas import tpu as pltpu


def _fused_gate_kernel(x_ref, w1t_ref, w2t_ref, o_ref):
    xb = x_ref[...]
    h = jax.lax.dot_general(
        xb, w1t_ref[...], (((1,), (0,)), ((), ())),
        preferred_element_type=jnp.float32,
    )
    h = jnp.maximum(h, 0.0)
    g = jax.lax.dot_general(
        h, w2t_ref[...], (((1,), (0,)), ((), ())),
        preferred_element_type=jnp.float32,
    )
    o_ref[...] = xb * jax.nn.sigmoid(g + g)


@functools.partial(jax.jit, static_argnames=("block_rows",))
def _run(x, w1t, w2t, block_rows):
    n, d = x.shape
    grid = (n // block_rows,)
    return pl.pallas_call(
        _fused_gate_kernel,
        grid=grid,
        in_specs=[
            pl.BlockSpec((block_rows, d), lambda i: (i, 0)),
            pl.BlockSpec((w1t.shape[0], w1t.shape[1]), lambda i: (0, 0)),
            pl.BlockSpec((w2t.shape[0], w2t.shape[1]), lambda i: (0, 0)),
        ],
        out_specs=pl.BlockSpec((block_rows, d), lambda i: (i, 0)),
        out_shape=jax.ShapeDtypeStruct((n, d), x.dtype),
        compiler_params=pltpu.CompilerParams(
            dimension_semantics=("parallel",),
        ),
    )(x, w1t, w2t)


def kernel(x, W1, W2):
    return _run(x, W1.T, W2.T, 6400)


# speedup vs baseline: 25.8746x; 1.0871x over previous
import functools

import jax
import jax.numpy as jnp
from jax.experimental import pallas as pl
from jax.experimental.pallas import tpu as pltpu


def _fused_gate_kernel(x_ref, w1t_ref, w2t_ref, o_ref):
    xb = x_ref[...]
    h = jax.lax.dot_general(
        xb, w1t_ref[...], (((1,), (0,)), ((), ())),
        preferred_element_type=jnp.float32,
    )
    h = jnp.maximum(h, 0.0)
    g = jax.lax.dot_general(
        h, w2t_ref[...], (((1,), (0,)), ((), ())),
        preferred_element_type=jnp.float32,
    )
    o_ref[...] = xb * jax.nn.sigmoid(g + g)


@functools.partial(jax.jit, static_argnames=("block_rows",))
def _run(x, w1t, w2t, block_rows):
    n, d = x.shape
    grid = (n // block_rows,)
    return pl.pallas_call(
        _fused_gate_kernel,
        grid=grid,
        in_specs=[
            pl.BlockSpec((block_rows, d), lambda i: (i, 0)),
            pl.BlockSpec((w1t.shape[0], w1t.shape[1]), lambda i: (0, 0)),
            pl.BlockSpec((w2t.shape[0], w2t.shape[1]), lambda i: (0, 0)),
        ],
        out_specs=pl.BlockSpec((block_rows, d), lambda i: (i, 0)),
        out_shape=jax.ShapeDtypeStruct((n, d), x.dtype),
        compiler_params=pltpu.CompilerParams(
            dimension_semantics=("parallel",),
        ),
    )(x, w1t, w2t)


def kernel(x, W1, W2):
    return _run(x, W1.T, W2.T, 12800)


# speedup vs baseline: 26.0151x; 1.0054x over previous
import functools

import jax
import jax.numpy as jnp
from jax.experimental import pallas as pl
from jax.experimental.pallas import tpu as pltpu


def _fused_gate_kernel(x_ref, w1t_ref, w2t_ref, o_ref):
    xb = x_ref[...]
    h = jax.lax.dot_general(
        xb, w1t_ref[...], (((1,), (0,)), ((), ())),
        preferred_element_type=jnp.float32,
    )
    h = jnp.maximum(h, 0.0)
    g = jax.lax.dot_general(
        h, w2t_ref[...], (((1,), (0,)), ((), ())),
        preferred_element_type=jnp.float32,
    )
    o_ref[...] = xb * jax.nn.sigmoid(g + g)


@functools.partial(jax.jit, static_argnames=("block_rows",))
def _run(x, w1t, w2t, block_rows):
    n, d = x.shape
    grid = (n // block_rows,)
    return pl.pallas_call(
        _fused_gate_kernel,
        grid=grid,
        in_specs=[
            pl.BlockSpec((block_rows, d), lambda i: (i, 0)),
            pl.BlockSpec((w1t.shape[0], w1t.shape[1]), lambda i: (0, 0)),
            pl.BlockSpec((w2t.shape[0], w2t.shape[1]), lambda i: (0, 0)),
        ],
        out_specs=pl.BlockSpec((block_rows, d), lambda i: (i, 0)),
        out_shape=jax.ShapeDtypeStruct((n, d), x.dtype),
        compiler_params=pltpu.CompilerParams(
            dimension_semantics=("parallel",),
        ),
    )(x, w1t, w2t)


def kernel(x, W1, W2):
    return _run(x, W1.T, W2.T, 20000)
